# Initial kernel scaffold; baseline (speedup 1.0000x reference)
import jax
import jax.numpy as jnp
from jax.experimental import pallas as pl


def kernel(x, token_table, pos_table):
    raise NotImplementedError("write your pallas kernel here")


# speedup vs baseline: 1.1769x; 1.1769x over previous
import functools

import jax
import jax.numpy as jnp
from jax import lax
from jax.experimental import pallas as pl
from jax.experimental.pallas import tpu as pltpu
from jax.experimental.pallas import tpu_sc as plsc

_VOCAB = 1000000
_MAXLEN = 200
_EMBED = 32
_BATCH = 4096

_NC = 2
_NS = 16
_NW = _NC * _NS

_N = _BATCH * _MAXLEN
_PER_W = _N // _NW
_SEQS_PER_CHUNK = 4
_CHUNK = _SEQS_PER_CHUNK * _MAXLEN
_NCHUNK = _PER_W // _CHUNK
_SUB = 80
_NSUB = _CHUNK // _SUB


def _tpe(xf, token_table, pos_table):
    mesh = plsc.VectorSubcoreMesh(core_axis_name="c", subcore_axis_name="s")

    @functools.partial(
        pl.kernel,
        out_type=jax.ShapeDtypeStruct((_N, _EMBED), jnp.float32),
        mesh=mesh,
        compiler_params=pltpu.CompilerParams(use_tc_tiling_on_sc=False),
        scratch_types=[
            pltpu.VMEM((_CHUNK,), jnp.int32),
            pltpu.VMEM((_CHUNK, _EMBED), jnp.float32),
            pltpu.VMEM((_MAXLEN, _EMBED), jnp.float32),
            pltpu.SemaphoreType.DMA,
        ],
    )
    def k(x_hbm, tok_hbm, pos_hbm, out_hbm, idx_v, rows_v, pos_v, sem):
        wid = lax.axis_index("s") * _NC + lax.axis_index("c")
        pltpu.sync_copy(pos_hbm, pos_v)
        base0 = wid * _PER_W

        def chunk_body(c, carry):
            base = base0 + c * _CHUNK
            pltpu.sync_copy(x_hbm.at[pl.ds(base, _CHUNK)], idx_v)
            copies = []
            for i in range(_NSUB):
                copies.append(
                    pltpu.async_copy(
                        tok_hbm.at[idx_v.at[pl.ds(i * _SUB, _SUB)]],
                        rows_v.at[pl.ds(i * _SUB, _SUB)],
                        sem,
                    )
                )
            for cp in copies:
                cp.wait()

            def row_body(r, carry2):
                m = lax.rem(r, _MAXLEN)
                rows_v[r, 0:16] = rows_v[r, 0:16] + pos_v[m, 0:16]
                rows_v[r, 16:32] = rows_v[r, 16:32] + pos_v[m, 16:32]
                return carry2

            lax.fori_loop(0, _CHUNK, row_body, 0, unroll=4)
            pltpu.sync_copy(rows_v, out_hbm.at[pl.ds(base, _CHUNK)])
            return carry

        lax.fori_loop(0, _NCHUNK, chunk_body, 0)

    return k(xf, token_table, pos_table)


def kernel(x, token_table, pos_table):
    xf = x.reshape(-1).astype(jnp.int32)
    out = _tpe(xf, token_table, pos_table)
    return out.reshape(x.shape[0], x.shape[1], _EMBED)


# speedup vs baseline: 1.3920x; 1.1827x over previous
import functools

import jax
import jax.numpy as jnp
from jax import lax
from jax.experimental import pallas as pl
from jax.experimental.pallas import tpu as pltpu
from jax.experimental.pallas import tpu_sc as plsc

_VOCAB = 1000000
_MAXLEN = 200
_EMBED = 32
_BATCH = 4096

_NC = 2
_NS = 16
_NW = _NC * _NS

_N = _BATCH * _MAXLEN
_PER_W = _N // _NW
_SEQS_PER_CHUNK = 4
_CHUNK = _SEQS_PER_CHUNK * _MAXLEN
_NCHUNK = _PER_W // _CHUNK
_SUB = 80
_NSUB = _CHUNK // _SUB


def _tpe(xf, token_table, pos_table):
    mesh = plsc.VectorSubcoreMesh(core_axis_name="c", subcore_axis_name="s")

    @functools.partial(
        pl.kernel,
        out_type=jax.ShapeDtypeStruct((_N, _EMBED), jnp.float32),
        mesh=mesh,
        compiler_params=pltpu.CompilerParams(use_tc_tiling_on_sc=False),
        scratch_types=[
            pltpu.VMEM((_CHUNK,), jnp.int32),
            pltpu.VMEM((_CHUNK, _EMBED), jnp.float32),
            pltpu.VMEM((_MAXLEN, _EMBED), jnp.float32),
            pltpu.SemaphoreType.DMA,
        ],
    )
    def k(x_hbm, tok_hbm, pos_hbm, out_hbm, idx_v, rows_v, pos_v, sem):
        wid = lax.axis_index("s") * _NC + lax.axis_index("c")
        pltpu.sync_copy(pos_hbm, pos_v)
        base0 = wid * _PER_W

        def chunk_body(c, carry):
            base = base0 + c * _CHUNK
            pltpu.sync_copy(x_hbm.at[pl.ds(base, _CHUNK)], idx_v)
            copies = []
            for i in range(_NSUB):
                copies.append(
                    pltpu.async_copy(
                        tok_hbm.at[idx_v.at[pl.ds(i * _SUB, _SUB)]],
                        rows_v.at[pl.ds(i * _SUB, _SUB)],
                        sem,
                    )
                )
            for cp in copies:
                cp.wait()

            def m_body(m, carry2):
                p0 = pos_v[m, 0:16]
                p1 = pos_v[m, 16:32]
                for s in range(_SEQS_PER_CHUNK):
                    r = s * _MAXLEN + m
                    plsc.addupdate(rows_v.at[r, pl.ds(0, 16)], p0)
                    plsc.addupdate(rows_v.at[r, pl.ds(16, 16)], p1)
                return carry2

            lax.fori_loop(0, _MAXLEN, m_body, 0, unroll=4)
            pltpu.sync_copy(rows_v, out_hbm.at[pl.ds(base, _CHUNK)])
            return carry

        lax.fori_loop(0, _NCHUNK, chunk_body, 0)

    return k(xf, token_table, pos_table)


def kernel(x, token_table, pos_table):
    xf = x.reshape(-1).astype(jnp.int32)
    out = _tpe(xf, token_table, pos_table)
    return out.reshape(x.shape[0], x.shape[1], _EMBED)


# speedup vs baseline: 1.3928x; 1.0006x over previous
import functools

import jax
import jax.numpy as jnp
from jax import lax
from jax.experimental import pallas as pl
from jax.experimental.pallas import tpu as pltpu
from jax.experimental.pallas import tpu_sc as plsc

_VOCAB = 1000000
_MAXLEN = 200
_EMBED = 32
_BATCH = 4096

_NC = 2
_NS = 16
_NW = _NC * _NS

_N = _BATCH * _MAXLEN
_PER_W = _N // _NW
_SEQS_PER_CHUNK = 4
_CHUNK = _SEQS_PER_CHUNK * _MAXLEN
_NCHUNK = _PER_W // _CHUNK
_SUB = 800
_NSUB = _CHUNK // _SUB


def _tpe(xf, token_table, pos_table):
    mesh = plsc.VectorSubcoreMesh(core_axis_name="c", subcore_axis_name="s")

    @functools.partial(
        pl.kernel,
        out_type=jax.ShapeDtypeStruct((_N, _EMBED), jnp.float32),
        mesh=mesh,
        compiler_params=pltpu.CompilerParams(use_tc_tiling_on_sc=False),
        scratch_types=[
            pltpu.VMEM((_CHUNK,), jnp.int32),
            pltpu.VMEM((_CHUNK, _EMBED), jnp.float32),
            pltpu.VMEM((_MAXLEN, _EMBED), jnp.float32),
            pltpu.SemaphoreType.DMA,
        ],
    )
    def k(x_hbm, tok_hbm, pos_hbm, out_hbm, idx_v, rows_v, pos_v, sem):
        wid = lax.axis_index("s") * _NC + lax.axis_index("c")
        pltpu.sync_copy(pos_hbm, pos_v)
        base0 = wid * _PER_W

        def chunk_body(c, carry):
            base = base0 + c * _CHUNK
            pltpu.sync_copy(x_hbm.at[pl.ds(base, _CHUNK)], idx_v)
            copies = []
            for i in range(_NSUB):
                copies.append(
                    pltpu.async_copy(
                        tok_hbm.at[idx_v.at[pl.ds(i * _SUB, _SUB)]],
                        rows_v.at[pl.ds(i * _SUB, _SUB)],
                        sem,
                    )
                )
            for cp in copies:
                cp.wait()

            def m_body(m, carry2):
                p0 = pos_v[m, 0:16]
                p1 = pos_v[m, 16:32]
                for s in range(_SEQS_PER_CHUNK):
                    r = s * _MAXLEN + m
                    plsc.addupdate(rows_v.at[r, pl.ds(0, 16)], p0)
                    plsc.addupdate(rows_v.at[r, pl.ds(16, 16)], p1)
                return carry2

            lax.fori_loop(0, _MAXLEN, m_body, 0, unroll=4)
            pltpu.sync_copy(rows_v, out_hbm.at[pl.ds(base, _CHUNK)])
            return carry

        lax.fori_loop(0, _NCHUNK, chunk_body, 0)

    return k(xf, token_table, pos_table)


def kernel(x, token_table, pos_table):
    xf = x.reshape(-1).astype(jnp.int32)
    out = _tpe(xf, token_table, pos_table)
    return out.reshape(x.shape[0], x.shape[1], _EMBED)
